# baseline (speedup 1.0000x reference)
import functools

import jax
import jax.numpy as jnp
from jax import lax
from jax.experimental import pallas as pl
from jax.experimental.pallas import tpu as pltpu
from jax.experimental.pallas import tpu_sc as plsc

_NUM_CORES = 2
_NUM_SUBCORES = 16
_NUM_WORKERS = _NUM_CORES * _NUM_SUBCORES

_CHUNK = 1600


@functools.lru_cache(maxsize=None)
def _build_gather(num_ids: int, vocab: int, dim: int):
    assert num_ids % _NUM_WORKERS == 0
    per_worker = num_ids // _NUM_WORKERS
    chunk = _CHUNK
    while per_worker % chunk:
        chunk //= 2
    n_chunks = per_worker // chunk

    mesh = plsc.VectorSubcoreMesh(
        core_axis_name="c", subcore_axis_name="s",
        num_cores=_NUM_CORES, num_subcores=_NUM_SUBCORES,
    )

    @functools.partial(
        pl.kernel,
        out_type=jax.ShapeDtypeStruct((num_ids, dim), jnp.float32),
        mesh=mesh,
        scratch_types=[
            pltpu.VMEM((chunk,), jnp.int32),
            pltpu.VMEM((chunk, dim), jnp.float32),
            pltpu.SemaphoreType.DMA,
        ],
        compiler_params=pltpu.CompilerParams(use_tc_tiling_on_sc=False),
    )
    def gather(ids_hbm, table_hbm, out_hbm, idx_v, rows_v, sem):
        wid = lax.axis_index("s") * _NUM_CORES + lax.axis_index("c")
        base = wid * per_worker

        @pl.loop(0, n_chunks)
        def _chunk_loop(i):
            off = base + i * chunk
            pltpu.sync_copy(ids_hbm.at[pl.ds(off, chunk)], idx_v)
            pltpu.async_copy(table_hbm.at[idx_v], rows_v, sem).wait()
            pltpu.sync_copy(rows_v, out_hbm.at[pl.ds(off, chunk)])

    return gather


def kernel(ids, table):
    vocab, dim = table.shape
    ids_flat = ids.reshape(-1)
    gather = _build_gather(ids_flat.shape[0], vocab, dim)
    out = gather(ids_flat, table)
    return out.reshape(ids.shape + (dim,))
